# baseline (speedup 1.0000x reference)
import functools

import jax
import jax.numpy as jnp
from jax.experimental import pallas as pl
from jax.experimental.pallas import tpu as pltpu

B = 4096
F_IN = 512
D = 10000
C = 100
HF = F_IN // 2

D_BLK = 4096
B_BLK = 2048
ND = (D + D_BLK - 1) // D_BLK
TAIL = D - (ND - 1) * D_BLK
NB = B // B_BLK

W_LANES = F_IN + 128


def _fused_kernel(enc_ref, cent_ref, s_ref, out_ref, g_ref, k_ref, csq_ref,
                  w_ref):
    t = pl.program_id(0)

    def stats(e, c):
        e8 = e.astype(jnp.float8_e4m3fn)
        cb = c.astype(jnp.bfloat16)
        gt = jax.lax.dot_general(e8[:, :HF], e8, (((0,), (0,)), ((), ())),
                                 preferred_element_type=jnp.float32)
        gbr = jax.lax.dot_general(e8[:, HF:], e8[:, HF:],
                                  (((0,), (0,)), ((), ())),
                                  preferred_element_type=jnp.float32)
        eb = e.astype(jnp.bfloat16)
        k = jax.lax.dot_general(cb, eb, (((1,), (0,)), ((), ())),
                                preferred_element_type=jnp.float32)
        return gt, gbr, k

    @pl.when(t == 0)
    def _init():
        c = cent_ref[...]
        gt, gbr, k = stats(enc_ref[...], c)
        g_ref[:HF, :] = gt
        g_ref[HF:, HF:] = gbr
        k_ref[...] = k
        csq_ref[...] = jnp.sum(c * c, axis=1, keepdims=True)

    @pl.when(jnp.logical_and(t > 0, t < ND - 1))
    def _accum():
        c = cent_ref[...]
        gt, gbr, k = stats(enc_ref[...], c)
        g_ref[:HF, :] += gt
        g_ref[HF:, HF:] += gbr
        k_ref[...] += k
        csq_ref[...] += jnp.sum(c * c, axis=1, keepdims=True)

    @pl.when(t == ND - 1)
    def _accum_tail():
        enc_ref[TAIL:, :] = jnp.zeros((D_BLK - TAIL, F_IN), jnp.float32)
        e = enc_ref[...]
        c = cent_ref[...]
        lanes = jax.lax.broadcasted_iota(jnp.int32, (1, D_BLK), 1)
        cm = jnp.where(lanes < TAIL, c, 0.0)
        gt, gbr, k = stats(e, cm)
        gt = g_ref[:HF, :] + gt
        gbr = g_ref[HF:, HF:] + gbr
        csq = csq_ref[...] + jnp.sum(cm * cm, axis=1, keepdims=True)
        cn = jnp.maximum(jnp.sqrt(csq), 1e-12)
        kp = (k_ref[...] + k) / cn
        gtb = gt.astype(jnp.bfloat16)
        w_ref[:HF, :F_IN] = gtb
        w_ref[HF:, :HF] = gtb[:, HF:].T
        w_ref[HF:, HF:F_IN] = gbr.astype(jnp.bfloat16)
        w_ref[:, F_IN:F_IN + C] = kp.astype(jnp.bfloat16).T

    @pl.when(t >= ND)
    def _scores():
        s = s_ref[...]
        sb = s.astype(jnp.bfloat16)
        tr = jnp.dot(sb, w_ref[...], preferred_element_type=jnp.float32)
        tt = tr[:, :F_IN]
        raw = tr[:, F_IN:F_IN + C]
        ssq = jnp.sum(tt * s, axis=1, keepdims=True)
        en = jnp.maximum(jnp.sqrt(ssq), 1e-12)
        out_ref[...] = raw / en


@functools.partial(jax.jit, static_argnames=("interpret",))
def kernel(samples, enc_weight, cent_weight, interpret=False):
    scores = pl.pallas_call(
        _fused_kernel,
        grid=(ND + NB,),
        in_specs=[
            pl.BlockSpec((D_BLK, F_IN), lambda t: (jnp.minimum(t, ND - 1), 0)),
            pl.BlockSpec((C, D_BLK), lambda t: (0, jnp.minimum(t, ND - 1))),
            pl.BlockSpec((B_BLK, F_IN), lambda t: (jnp.maximum(t - ND, 0), 0)),
        ],
        out_specs=pl.BlockSpec((B_BLK, C), lambda t: (jnp.maximum(t - ND, 0), 0)),
        out_shape=jax.ShapeDtypeStruct((B, C), jnp.float32),
        scratch_shapes=[
            pltpu.VMEM((F_IN, F_IN), jnp.float32),
            pltpu.VMEM((C, F_IN), jnp.float32),
            pltpu.VMEM((C, 1), jnp.float32),
            pltpu.VMEM((F_IN, W_LANES), jnp.bfloat16),
        ],
        interpret=interpret,
    )(enc_weight, cent_weight, samples)
    return scores


# speedup vs baseline: 1.0207x; 1.0207x over previous
import functools

import jax
import jax.numpy as jnp
from jax.experimental import pallas as pl
from jax.experimental.pallas import tpu as pltpu

B = 4096
F_IN = 512
D = 10000
C = 100
HF = F_IN // 2

D_BLK = 5120
B_BLK = 2048
ND = (D + D_BLK - 1) // D_BLK
TAIL = D - (ND - 1) * D_BLK
NB = B // B_BLK

W_LANES = F_IN + 128


def _fused_kernel(enc_ref, cent_ref, s_ref, out_ref, g_ref, k_ref, csq_ref,
                  w_ref):
    t = pl.program_id(0)

    def stats(e, c):
        e8 = e.astype(jnp.float8_e4m3fn)
        cb = c.astype(jnp.bfloat16)
        gt = jax.lax.dot_general(e8[:, :HF], e8, (((0,), (0,)), ((), ())),
                                 preferred_element_type=jnp.float32)
        gbr = jax.lax.dot_general(e8[:, HF:], e8[:, HF:],
                                  (((0,), (0,)), ((), ())),
                                  preferred_element_type=jnp.float32)
        eb = e.astype(jnp.bfloat16)
        k = jax.lax.dot_general(cb, eb, (((1,), (0,)), ((), ())),
                                preferred_element_type=jnp.float32)
        return gt, gbr, k

    @pl.when(t == 0)
    def _init():
        c = cent_ref[...]
        gt, gbr, k = stats(enc_ref[...], c)
        g_ref[:HF, :] = gt
        g_ref[HF:, HF:] = gbr
        k_ref[...] = k
        csq_ref[...] = jnp.sum(c * c, axis=1, keepdims=True)

    @pl.when(jnp.logical_and(t > 0, t < ND - 1))
    def _accum():
        c = cent_ref[...]
        gt, gbr, k = stats(enc_ref[...], c)
        g_ref[:HF, :] += gt
        g_ref[HF:, HF:] += gbr
        k_ref[...] += k
        csq_ref[...] += jnp.sum(c * c, axis=1, keepdims=True)

    @pl.when(t == ND - 1)
    def _accum_tail():
        enc_ref[TAIL:, :] = jnp.zeros((D_BLK - TAIL, F_IN), jnp.float32)
        e = enc_ref[...]
        c = cent_ref[...]
        lanes = jax.lax.broadcasted_iota(jnp.int32, (1, D_BLK), 1)
        cm = jnp.where(lanes < TAIL, c, 0.0)
        gt, gbr, k = stats(e, cm)
        gt = g_ref[:HF, :] + gt
        gbr = g_ref[HF:, HF:] + gbr
        csq = csq_ref[...] + jnp.sum(cm * cm, axis=1, keepdims=True)
        cn = jnp.maximum(jnp.sqrt(csq), 1e-12)
        kp = (k_ref[...] + k) / cn
        gtb = gt.astype(jnp.bfloat16)
        w_ref[:HF, :F_IN] = gtb
        w_ref[HF:, :HF] = gtb[:, HF:].T
        w_ref[HF:, HF:F_IN] = gbr.astype(jnp.bfloat16)
        w_ref[:, F_IN:F_IN + C] = kp.astype(jnp.bfloat16).T

    @pl.when(t >= ND)
    def _scores():
        s = s_ref[...]
        sb = s.astype(jnp.bfloat16)
        tr = jnp.dot(sb, w_ref[...], preferred_element_type=jnp.float32)
        tt = tr[:, :F_IN]
        raw = tr[:, F_IN:F_IN + C]
        ssq = jnp.sum(tt * s, axis=1, keepdims=True)
        en = jnp.maximum(jnp.sqrt(ssq), 1e-12)
        out_ref[...] = raw / en


@functools.partial(jax.jit, static_argnames=("interpret",))
def kernel(samples, enc_weight, cent_weight, interpret=False):
    scores = pl.pallas_call(
        _fused_kernel,
        grid=(ND + NB,),
        in_specs=[
            pl.BlockSpec((D_BLK, F_IN), lambda t: (jnp.minimum(t, ND - 1), 0)),
            pl.BlockSpec((C, D_BLK), lambda t: (0, jnp.minimum(t, ND - 1))),
            pl.BlockSpec((B_BLK, F_IN), lambda t: (jnp.maximum(t - ND, 0), 0)),
        ],
        out_specs=pl.BlockSpec((B_BLK, C), lambda t: (jnp.maximum(t - ND, 0), 0)),
        out_shape=jax.ShapeDtypeStruct((B, C), jnp.float32),
        scratch_shapes=[
            pltpu.VMEM((F_IN, F_IN), jnp.float32),
            pltpu.VMEM((C, F_IN), jnp.float32),
            pltpu.VMEM((C, 1), jnp.float32),
            pltpu.VMEM((F_IN, W_LANES), jnp.bfloat16),
        ],
        interpret=interpret,
    )(enc_weight, cent_weight, samples)
    return scores
